# Initial kernel scaffold; baseline (speedup 1.0000x reference)
import jax
import jax.numpy as jnp
from jax.experimental import pallas as pl


def kernel(x, edge_index, edge_attr, batch, params):
    raise NotImplementedError("write your pallas kernel here")


# speedup vs baseline: 1.9260x; 1.9260x over previous
import functools

import jax
import jax.numpy as jnp
from jax import lax
from jax.experimental import pallas as pl
from jax.experimental.pallas import tpu as pltpu
from jax.experimental.pallas import tpu_sc as plsc

F32 = jnp.float32


def _sigm(x):
    return 1.0 / (1.0 + jnp.exp(-x))


def _leaky(x):
    return jnp.where(x > 0, x, 0.01 * x)


def _elu(x):
    return jnp.where(x > 0, x, jnp.exp(jnp.minimum(x, 0.0)) - 1.0)


def _gru_math(inp, hid, wiht, biht, whht, bhht):
    H = hid.shape[1]
    gi = jnp.dot(inp, wiht, preferred_element_type=F32) + biht
    gh = jnp.dot(hid, whht, preferred_element_type=F32) + bhht
    r = _sigm(gi[:, :H] + gh[:, :H])
    z = _sigm(gi[:, H:2 * H] + gh[:, H:2 * H])
    n = jnp.tanh(gi[:, 2 * H:] + r * gh[:, 2 * H:])
    return (1.0 - z) * n + z * hid


def _proj_out(x1, wkt, bk, wqt, bq, wvt, bv, wst, gb, k_ref, qv_ref, s_ref):
    H = k_ref.shape[1]
    k_ref[...] = jnp.dot(x1, wkt[...], preferred_element_type=F32) + bk[...]
    qv_ref[:, :H] = jnp.dot(x1, wqt[...], preferred_element_type=F32) + bq[...]
    qv_ref[:, H:] = jnp.dot(x1, wvt[...], preferred_element_type=F32) + bv[...]
    s_ref[...] = jnp.dot(x1, wst[...], preferred_element_type=F32) + gb[...]


def _tc_pre_body(x_ref, w1t, b1, wkt, bk, wqt, bq, wvt, bv, wst, gb,
                 x1_ref, k_ref, qv_ref, s_ref):
    x = x_ref[...]
    x1 = _leaky(jnp.dot(x, w1t[...], preferred_element_type=F32) + b1[...])
    x1_ref[...] = x1
    _proj_out(x1, wkt, bk, wqt, bq, wvt, bv, wst, gb, k_ref, qv_ref, s_ref)


def _tc_mid_body(agg_ref, s_ref, xp_ref, wiht, biht, whht, bhht,
                 wkt, bk, wqt, bq, wvt, bv, wst, gb,
                 x2_ref, k_ref, qv_ref, s2_ref):
    xp = xp_ref[...]
    agg = agg_ref[0] + agg_ref[1]
    h = _elu(agg + s_ref[...])
    x2 = jnp.maximum(_gru_math(h, xp, wiht[...], biht[...], whht[...], bhht[...]), 0.0)
    x2_ref[...] = x2
    _proj_out(x2, wkt, bk, wqt, bq, wvt, bv, wst, gb, k_ref, qv_ref, s2_ref)


def _tc_post_body(agg_ref, s_ref, xp_ref, wiht, biht, whht, bhht, wgt, asrc,
                  x3_ref, xs_ref, as_ref):
    xp = xp_ref[...]
    agg = agg_ref[0] + agg_ref[1]
    h = _elu(agg + s_ref[...])
    x3 = jnp.maximum(_gru_math(h, xp, wiht[...], biht[...], whht[...], bhht[...]), 0.0)
    x3_ref[...] = x3
    xs = jnp.dot(x3, wgt[...], preferred_element_type=F32)
    xs_ref[...] = xs
    as_ref[...] = jnp.dot(xs, asrc[...], preferred_element_type=F32)


def _tc_gat_body(T, x3_ref, xs_ref, asrc_ref, bcol_ref, brow_ref,
                 wgt, adst, gbias, wiht, biht, whht, bhht, w2t, b2, y_ref):
    Bg = y_ref.shape[0]
    bcol = bcol_ref[...]
    brow = brow_ref[...]
    st_bool = bcol == lax.broadcasted_iota(jnp.int32, (1, Bg), 1)
    stf = st_bool.astype(F32)
    sf = (lax.broadcasted_iota(jnp.int32, (Bg, 1), 0) == brow).astype(F32)
    x3 = x3_ref[...]
    xs = xs_ref[...]
    a_src = asrc_ref[...]
    out = jnp.maximum(jnp.dot(sf, x3, preferred_element_type=F32), 0.0)
    for _ in range(T):
        od = jnp.dot(out, wgt[...], preferred_element_type=F32)
        a_dst = jnp.dot(od, adst[...], preferred_element_type=F32)
        alpha = _leaky(a_src + jnp.dot(stf, a_dst, preferred_element_type=F32))
        masked = jnp.where(st_bool, alpha, -3.0e38)
        amax = jnp.max(masked, axis=0, keepdims=True)
        ex = jnp.exp(alpha - jnp.dot(stf, amax.T, preferred_element_type=F32))
        den = jnp.dot(sf, ex, preferred_element_type=F32)
        att = ex / (jnp.dot(stf, den, preferred_element_type=F32) + 1e-16)
        m = jnp.dot(sf, att * xs, preferred_element_type=F32) + gbias[...]
        h = _elu(m)
        out = jnp.maximum(
            _gru_math(h, out, wiht[...], biht[...], whht[...], bhht[...]), 0.0)
    y_ref[...] = jnp.dot(out, w2t[...], preferred_element_type=F32) + b2[...]


def _row_spec(rb, d):
    return pl.BlockSpec((rb, d), lambda i: (i, 0))


def _full_spec(shape):
    nd = len(shape)
    return pl.BlockSpec(shape, lambda i: (0,) * nd)


def _make_edge_agg(N, H, E, C, CHW):
    NT = 16
    ZR = 40
    NCH = N // ZR
    NRR = (NCH + NT - 1) // NT
    mesh = plsc.VectorSubcoreMesh(core_axis_name="c", subcore_axis_name="s")

    @functools.partial(
        pl.kernel,
        out_type=jax.ShapeDtypeStruct((2 * N, H), F32),
        mesh=mesh,
        scratch_types=[
            pltpu.VMEM((C,), jnp.int32),
            pltpu.VMEM((C,), jnp.int32),
            pltpu.VMEM((C, H), F32),
            pltpu.VMEM((C, 2 * H), F32),
            pltpu.VMEM((C, H), F32),
            pltpu.VMEM((ZR, H), F32),
            pltpu.VMEM_SHARED((N, H), F32),
            pltpu.SemaphoreType.DMA,
            pltpu.SemaphoreType.DMA,
        ],
    )
    def edge_agg(k_hbm, qv_hbm, src_hbm, dst_hbm, out_hbm,
                 src_c, dst_c, krows, qvrows, msg, zbuf, acc, sem1, sem2):
        c = lax.axis_index("c")
        s = lax.axis_index("s")
        w = c * NT + s

        zero = jnp.zeros((16,), F32)

        def zrow(i, carry):
            for l in range(H // 16):
                zbuf[i, pl.ds(l * 16, 16)] = zero
            return carry

        lax.fori_loop(0, ZR, zrow, 0)
        for r in range(NRR):
            cid = s + NT * r

            @pl.when(cid < NCH)
            def _():
                pltpu.sync_copy(zbuf, acc.at[pl.ds(cid * ZR, ZR)])
        plsc.subcore_barrier()

        def chunk(j, carry):
            flat0 = (w * CHW + j) * C
            pltpu.sync_copy(src_hbm.at[pl.ds(flat0, C)], src_c)
            pltpu.sync_copy(dst_hbm.at[pl.ds(flat0, C)], dst_c)
            cp1 = pltpu.async_copy(k_hbm.at[dst_c], krows, sem1)
            cp2 = pltpu.async_copy(qv_hbm.at[src_c], qvrows, sem2)
            cp1.wait()
            cp2.wait()

            def edge(e, ecarry):
                real = flat0 + e < E
                for l in range(H // 16):
                    kk = krows[e, pl.ds(l * 16, 16)]
                    qq = qvrows[e, pl.ds(l * 16, 16)]
                    vv = qvrows[e, pl.ds(H + l * 16, 16)]
                    g = 1.0 / (1.0 + jnp.exp(-(kk + qq)))
                    msg[e, pl.ds(l * 16, 16)] = jnp.where(real, g * vv, 0.0)
                return ecarry

            lax.fori_loop(0, C, edge, 0)
            pltpu.sync_copy(msg, acc.at[dst_c], add=True)
            return carry

        lax.fori_loop(0, CHW, chunk, 0)

        plsc.subcore_barrier()
        for r in range(NRR):
            cid = s + NT * r

            @pl.when(cid < NCH)
            def _():
                pltpu.sync_copy(acc.at[pl.ds(cid * ZR, ZR)], zbuf)
                pltpu.sync_copy(zbuf, out_hbm.at[pl.ds(c * N + cid * ZR, ZR)])

    return edge_agg


def kernel(x, edge_index, edge_attr, batch, params):
    N, IN = x.shape
    E = edge_index.shape[1]
    B = 256
    H = params['W1'].shape[0]
    T = 8
    RB = 2000
    NROWS = N // RB
    C = 64
    W = 32
    CHW = -(-E // (W * C))
    EPAD = W * CHW * C - E

    g1, g2 = params['g1'], params['g2']
    gat = params['gat']
    pad = jnp.zeros((EPAD,), jnp.int32)
    src2 = jnp.concatenate([edge_index[0], pad])
    dst2 = jnp.concatenate([edge_index[1], pad])
    bcol = batch.reshape(N, 1)
    brow = batch.reshape(1, N)

    rs = functools.partial(_row_spec, RB)
    wspecs = [_full_spec(s) for s in
              ((IN, H), (1, H), (H, H), (1, H), (H, H), (1, H), (H, H), (1, H),
               (H, H), (1, H))]
    proj_outspecs = [rs(H), rs(H), rs(2 * H), rs(H)]
    proj_outshapes = [jax.ShapeDtypeStruct((N, H), F32),
                      jax.ShapeDtypeStruct((N, H), F32),
                      jax.ShapeDtypeStruct((N, 2 * H), F32),
                      jax.ShapeDtypeStruct((N, H), F32)]
    x1, k1, qv1, s1 = pl.pallas_call(
        _tc_pre_body,
        grid=(NROWS,),
        in_specs=[rs(IN)] + wspecs,
        out_specs=proj_outspecs,
        out_shape=proj_outshapes,
    )(x, params['W1'].T, params['b1'].reshape(1, H),
      g1['Wk'].T, g1['bk'].reshape(1, H), g1['Wq'].T, g1['bq'].reshape(1, H),
      g1['Wv'].T, g1['bv'].reshape(1, H), g1['Wskip'].T, g1['bias'].reshape(1, H))

    edge_agg = _make_edge_agg(N, H, E, C, CHW)

    agg1 = edge_agg(k1, qv1, src2, dst2).reshape(2, N, H)

    gru0 = params['gru0']
    aggspec = pl.BlockSpec((2, RB, H), lambda i: (0, i, 0))
    gruspecs = [_full_spec(s) for s in
                ((H, 3 * H), (1, 3 * H), (H, 3 * H), (1, 3 * H))]
    x2, k2, qv2, s2 = pl.pallas_call(
        _tc_mid_body,
        grid=(NROWS,),
        in_specs=[aggspec, rs(H), rs(H)] + gruspecs + wspecs[2:],
        out_specs=proj_outspecs,
        out_shape=proj_outshapes,
    )(agg1, s1, x1,
      gru0['Wih'].T, gru0['bih'].reshape(1, 3 * H),
      gru0['Whh'].T, gru0['bhh'].reshape(1, 3 * H),
      g2['Wk'].T, g2['bk'].reshape(1, H), g2['Wq'].T, g2['bq'].reshape(1, H),
      g2['Wv'].T, g2['bv'].reshape(1, H), g2['Wskip'].T, g2['bias'].reshape(1, H))

    agg2 = edge_agg(k2, qv2, src2, dst2).reshape(2, N, H)

    gru1 = params['gru1']
    x3, xs, a_src = pl.pallas_call(
        _tc_post_body,
        grid=(NROWS,),
        in_specs=[aggspec, rs(H), rs(H)] + gruspecs +
                 [_full_spec((H, H)), _full_spec((H, 1))],
        out_specs=[rs(H), rs(H), rs(1)],
        out_shape=[jax.ShapeDtypeStruct((N, H), F32),
                   jax.ShapeDtypeStruct((N, H), F32),
                   jax.ShapeDtypeStruct((N, 1), F32)],
    )(agg2, s2, x2,
      gru1['Wih'].T, gru1['bih'].reshape(1, 3 * H),
      gru1['Whh'].T, gru1['bhh'].reshape(1, 3 * H),
      gat['Wg'].T, gat['att_src'].reshape(H, 1))

    mg = params['molgru']
    y = pl.pallas_call(
        functools.partial(_tc_gat_body, T),
        out_shape=jax.ShapeDtypeStruct((B, 1), F32),
    )(x3, xs, a_src, bcol, brow,
      gat['Wg'].T, gat['att_dst'].reshape(H, 1), gat['bias'].reshape(1, H),
      mg['Wih'].T, mg['bih'].reshape(1, 3 * H),
      mg['Whh'].T, mg['bhh'].reshape(1, 3 * H),
      params['W2'].T, params['b2'].reshape(1, 1))
    return y


# speedup vs baseline: 2.0164x; 1.0470x over previous
import functools

import jax
import jax.numpy as jnp
from jax import lax
from jax.experimental import pallas as pl
from jax.experimental.pallas import tpu as pltpu
from jax.experimental.pallas import tpu_sc as plsc

F32 = jnp.float32


def _sigm(x):
    return 1.0 / (1.0 + jnp.exp(-x))


def _leaky(x):
    return jnp.where(x > 0, x, 0.01 * x)


def _elu(x):
    return jnp.where(x > 0, x, jnp.exp(jnp.minimum(x, 0.0)) - 1.0)


def _gru_math(inp, hid, wiht, biht, whht, bhht):
    H = hid.shape[1]
    gi = jnp.dot(inp, wiht, preferred_element_type=F32) + biht
    gh = jnp.dot(hid, whht, preferred_element_type=F32) + bhht
    r = _sigm(gi[:, :H] + gh[:, :H])
    z = _sigm(gi[:, H:2 * H] + gh[:, H:2 * H])
    n = jnp.tanh(gi[:, 2 * H:] + r * gh[:, 2 * H:])
    return (1.0 - z) * n + z * hid


def _proj_out(x1, wkt, bk, wqt, bq, wvt, bv, wst, gb, k_ref, qv_ref, s_ref):
    H = k_ref.shape[1]
    k_ref[...] = jnp.dot(x1, wkt[...], preferred_element_type=F32) + bk[...]
    qv_ref[:, :H] = jnp.dot(x1, wqt[...], preferred_element_type=F32) + bq[...]
    qv_ref[:, H:] = jnp.dot(x1, wvt[...], preferred_element_type=F32) + bv[...]
    s_ref[...] = jnp.dot(x1, wst[...], preferred_element_type=F32) + gb[...]


def _tc_pre_body(x_ref, w1t, b1, wkt, bk, wqt, bq, wvt, bv, wst, gb,
                 x1_ref, k_ref, qv_ref, s_ref):
    x = x_ref[...]
    x1 = _leaky(jnp.dot(x, w1t[...], preferred_element_type=F32) + b1[...])
    x1_ref[...] = x1
    _proj_out(x1, wkt, bk, wqt, bq, wvt, bv, wst, gb, k_ref, qv_ref, s_ref)


def _tc_mid_body(agg_ref, s_ref, xp_ref, wiht, biht, whht, bhht,
                 wkt, bk, wqt, bq, wvt, bv, wst, gb,
                 x2_ref, k_ref, qv_ref, s2_ref):
    xp = xp_ref[...]
    agg = agg_ref[0] + agg_ref[1]
    h = _elu(agg + s_ref[...])
    x2 = jnp.maximum(_gru_math(h, xp, wiht[...], biht[...], whht[...], bhht[...]), 0.0)
    x2_ref[...] = x2
    _proj_out(x2, wkt, bk, wqt, bq, wvt, bv, wst, gb, k_ref, qv_ref, s2_ref)


def _tc_post_body(agg_ref, s_ref, xp_ref, wiht, biht, whht, bhht, wgt, asrc,
                  x3_ref, xs_ref, as_ref):
    xp = xp_ref[...]
    agg = agg_ref[0] + agg_ref[1]
    h = _elu(agg + s_ref[...])
    x3 = jnp.maximum(_gru_math(h, xp, wiht[...], biht[...], whht[...], bhht[...]), 0.0)
    x3_ref[...] = x3
    xs = jnp.dot(x3, wgt[...], preferred_element_type=F32)
    xs_ref[...] = xs
    as_ref[...] = jnp.dot(xs, asrc[...], preferred_element_type=F32)


def _tc_gat_body(T, x3_ref, xs_ref, asrc_ref, bcol_ref, brow_ref,
                 wgt, adst, gbias, wiht, biht, whht, bhht, w2t, b2, y_ref):
    Bg = y_ref.shape[0]
    bcol = bcol_ref[...]
    brow = brow_ref[...]
    st_bool = bcol == lax.broadcasted_iota(jnp.int32, (1, Bg), 1)
    stf = st_bool.astype(F32)
    sf = (lax.broadcasted_iota(jnp.int32, (Bg, 1), 0) == brow).astype(F32)
    x3 = x3_ref[...]
    xs = xs_ref[...]
    a_src = asrc_ref[...]
    out = jnp.maximum(jnp.dot(sf, x3, preferred_element_type=F32), 0.0)
    for _ in range(T):
        od = jnp.dot(out, wgt[...], preferred_element_type=F32)
        a_dst = jnp.dot(od, adst[...], preferred_element_type=F32)
        alpha = _leaky(a_src + jnp.dot(stf, a_dst, preferred_element_type=F32))
        masked = jnp.where(st_bool, alpha, -3.0e38)
        amax = jnp.max(masked, axis=0, keepdims=True)
        ex = jnp.exp(alpha - jnp.dot(stf, amax.T, preferred_element_type=F32))
        den = jnp.dot(sf, ex, preferred_element_type=F32)
        att = ex / (jnp.dot(stf, den, preferred_element_type=F32) + 1e-16)
        m = jnp.dot(sf, att * xs, preferred_element_type=F32) + gbias[...]
        h = _elu(m)
        out = jnp.maximum(
            _gru_math(h, out, wiht[...], biht[...], whht[...], bhht[...]), 0.0)
    y_ref[...] = jnp.dot(out, w2t[...], preferred_element_type=F32) + b2[...]


def _row_spec(rb, d):
    return pl.BlockSpec((rb, d), lambda i: (i, 0))


def _full_spec(shape):
    nd = len(shape)
    return pl.BlockSpec(shape, lambda i: (0,) * nd)


def _make_edge_agg(N, H, E, C, CHW):
    NT = 16
    ZR = 40
    NCH = N // ZR
    NRR = (NCH + NT - 1) // NT
    mesh = plsc.VectorSubcoreMesh(core_axis_name="c", subcore_axis_name="s")

    @functools.partial(
        pl.kernel,
        out_type=jax.ShapeDtypeStruct((2 * N, H), F32),
        mesh=mesh,
        scratch_types=[
            pltpu.VMEM((C,), jnp.int32),
            pltpu.VMEM((C,), jnp.int32),
            pltpu.VMEM((C, H), F32),
            pltpu.VMEM((C, 2 * H), F32),
            pltpu.VMEM((C, H), F32),
            pltpu.VMEM((ZR, H), F32),
            pltpu.VMEM_SHARED((N, H), F32),
            pltpu.SemaphoreType.DMA,
            pltpu.SemaphoreType.DMA,
        ],
    )
    def edge_agg(k_hbm, qv_hbm, src_hbm, dst_hbm, out_hbm,
                 src_c, dst_c, krows, qvrows, msg, zbuf, acc, sem1, sem2):
        c = lax.axis_index("c")
        s = lax.axis_index("s")
        w = c * NT + s

        zero = jnp.zeros((16,), F32)

        def zrow(i, carry):
            for l in range(H // 16):
                zbuf[i, pl.ds(l * 16, 16)] = zero
            return carry

        lax.fori_loop(0, ZR, zrow, 0)
        for r in range(NRR):
            cid = s + NT * r

            @pl.when(cid < NCH)
            def _():
                pltpu.sync_copy(zbuf, acc.at[pl.ds(cid * ZR, ZR)])
        plsc.subcore_barrier()

        base = w * CHW * C
        nreal = jnp.clip(E - base, 0, CHW * C)
        nfull = nreal // C
        ntot = (nreal + C - 1) // C

        def make_chunk(masked):
            def chunk(j, carry):
                flat0 = base + j * C
                pltpu.sync_copy(src_hbm.at[pl.ds(flat0, C)], src_c)
                pltpu.sync_copy(dst_hbm.at[pl.ds(flat0, C)], dst_c)
                cp1 = pltpu.async_copy(k_hbm.at[dst_c], krows, sem1)
                cp2 = pltpu.async_copy(qv_hbm.at[src_c], qvrows, sem2)
                cp1.wait()
                cp2.wait()

                def edge(e0, ecarry):
                    for u in range(4):
                        e = e0 * 4 + u
                        for l in range(H // 16):
                            kk = krows[e, pl.ds(l * 16, 16)]
                            qq = qvrows[e, pl.ds(l * 16, 16)]
                            vv = qvrows[e, pl.ds(H + l * 16, 16)]
                            g = 1.0 / (1.0 + jnp.exp(-(kk + qq)))
                            m = g * vv
                            if masked:
                                m = jnp.where(flat0 + e < E, m, 0.0)
                            msg[e, pl.ds(l * 16, 16)] = m
                    return ecarry

                lax.fori_loop(0, C // 4, edge, 0)
                pltpu.sync_copy(msg, acc.at[dst_c], add=True)
                return carry

            return chunk

        lax.fori_loop(0, nfull, make_chunk(False), 0)
        lax.fori_loop(nfull, ntot, make_chunk(True), 0)

        plsc.subcore_barrier()
        for r in range(NRR):
            cid = s + NT * r

            @pl.when(cid < NCH)
            def _():
                pltpu.sync_copy(acc.at[pl.ds(cid * ZR, ZR)], zbuf)
                pltpu.sync_copy(zbuf, out_hbm.at[pl.ds(c * N + cid * ZR, ZR)])

    return edge_agg


def kernel(x, edge_index, edge_attr, batch, params):
    N, IN = x.shape
    E = edge_index.shape[1]
    B = 256
    H = params['W1'].shape[0]
    T = 8
    RB = 2000
    NROWS = N // RB
    C = 64
    W = 32
    CHW = -(-E // (W * C))
    EPAD = W * CHW * C - E

    g1, g2 = params['g1'], params['g2']
    gat = params['gat']
    pad = jnp.zeros((EPAD,), jnp.int32)
    src2 = jnp.concatenate([edge_index[0], pad])
    dst2 = jnp.concatenate([edge_index[1], pad])
    bcol = batch.reshape(N, 1)
    brow = batch.reshape(1, N)

    rs = functools.partial(_row_spec, RB)
    wspecs = [_full_spec(s) for s in
              ((IN, H), (1, H), (H, H), (1, H), (H, H), (1, H), (H, H), (1, H),
               (H, H), (1, H))]
    proj_outspecs = [rs(H), rs(H), rs(2 * H), rs(H)]
    proj_outshapes = [jax.ShapeDtypeStruct((N, H), F32),
                      jax.ShapeDtypeStruct((N, H), F32),
                      jax.ShapeDtypeStruct((N, 2 * H), F32),
                      jax.ShapeDtypeStruct((N, H), F32)]
    x1, k1, qv1, s1 = pl.pallas_call(
        _tc_pre_body,
        grid=(NROWS,),
        in_specs=[rs(IN)] + wspecs,
        out_specs=proj_outspecs,
        out_shape=proj_outshapes,
    )(x, params['W1'].T, params['b1'].reshape(1, H),
      g1['Wk'].T, g1['bk'].reshape(1, H), g1['Wq'].T, g1['bq'].reshape(1, H),
      g1['Wv'].T, g1['bv'].reshape(1, H), g1['Wskip'].T, g1['bias'].reshape(1, H))

    edge_agg = _make_edge_agg(N, H, E, C, CHW)

    agg1 = edge_agg(k1, qv1, src2, dst2).reshape(2, N, H)

    gru0 = params['gru0']
    aggspec = pl.BlockSpec((2, RB, H), lambda i: (0, i, 0))
    gruspecs = [_full_spec(s) for s in
                ((H, 3 * H), (1, 3 * H), (H, 3 * H), (1, 3 * H))]
    x2, k2, qv2, s2 = pl.pallas_call(
        _tc_mid_body,
        grid=(NROWS,),
        in_specs=[aggspec, rs(H), rs(H)] + gruspecs + wspecs[2:],
        out_specs=proj_outspecs,
        out_shape=proj_outshapes,
    )(agg1, s1, x1,
      gru0['Wih'].T, gru0['bih'].reshape(1, 3 * H),
      gru0['Whh'].T, gru0['bhh'].reshape(1, 3 * H),
      g2['Wk'].T, g2['bk'].reshape(1, H), g2['Wq'].T, g2['bq'].reshape(1, H),
      g2['Wv'].T, g2['bv'].reshape(1, H), g2['Wskip'].T, g2['bias'].reshape(1, H))

    agg2 = edge_agg(k2, qv2, src2, dst2).reshape(2, N, H)

    gru1 = params['gru1']
    x3, xs, a_src = pl.pallas_call(
        _tc_post_body,
        grid=(NROWS,),
        in_specs=[aggspec, rs(H), rs(H)] + gruspecs +
                 [_full_spec((H, H)), _full_spec((H, 1))],
        out_specs=[rs(H), rs(H), rs(1)],
        out_shape=[jax.ShapeDtypeStruct((N, H), F32),
                   jax.ShapeDtypeStruct((N, H), F32),
                   jax.ShapeDtypeStruct((N, 1), F32)],
    )(agg2, s2, x2,
      gru1['Wih'].T, gru1['bih'].reshape(1, 3 * H),
      gru1['Whh'].T, gru1['bhh'].reshape(1, 3 * H),
      gat['Wg'].T, gat['att_src'].reshape(H, 1))

    mg = params['molgru']
    y = pl.pallas_call(
        functools.partial(_tc_gat_body, T),
        out_shape=jax.ShapeDtypeStruct((B, 1), F32),
    )(x3, xs, a_src, bcol, brow,
      gat['Wg'].T, gat['att_dst'].reshape(H, 1), gat['bias'].reshape(1, H),
      mg['Wih'].T, mg['bih'].reshape(1, 3 * H),
      mg['Whh'].T, mg['bhh'].reshape(1, 3 * H),
      params['W2'].T, params['b2'].reshape(1, 1))
    return y


# speedup vs baseline: 2.1175x; 1.0501x over previous
import functools

import jax
import jax.numpy as jnp
from jax import lax
from jax.experimental import pallas as pl
from jax.experimental.pallas import tpu as pltpu
from jax.experimental.pallas import tpu_sc as plsc

F32 = jnp.float32


def _sigm(x):
    return 1.0 / (1.0 + jnp.exp(-x))


def _leaky(x):
    return jnp.where(x > 0, x, 0.01 * x)


def _elu(x):
    return jnp.where(x > 0, x, jnp.exp(jnp.minimum(x, 0.0)) - 1.0)


def _gru_math(inp, hid, wiht, biht, whht, bhht):
    H = hid.shape[1]
    gi = jnp.dot(inp, wiht, preferred_element_type=F32) + biht
    gh = jnp.dot(hid, whht, preferred_element_type=F32) + bhht
    r = _sigm(gi[:, :H] + gh[:, :H])
    z = _sigm(gi[:, H:2 * H] + gh[:, H:2 * H])
    n = jnp.tanh(gi[:, 2 * H:] + r * gh[:, 2 * H:])
    return (1.0 - z) * n + z * hid


def _proj_out(x1, wkt, bk, wqt, bq, wvt, bv, wst, gb, k_ref, qv_ref, s_ref):
    H = k_ref.shape[1]
    k_ref[...] = jnp.dot(x1, wkt[...], preferred_element_type=F32) + bk[...]
    qv_ref[:, :H] = jnp.dot(x1, wqt[...], preferred_element_type=F32) + bq[...]
    qv_ref[:, H:] = jnp.dot(x1, wvt[...], preferred_element_type=F32) + bv[...]
    s_ref[...] = jnp.dot(x1, wst[...], preferred_element_type=F32) + gb[...]


def _tc_pre_body(x_ref, w1t, b1, wkt, bk, wqt, bq, wvt, bv, wst, gb,
                 x1_ref, k_ref, qv_ref, s_ref):
    x = x_ref[...]
    x1 = _leaky(jnp.dot(x, w1t[...], preferred_element_type=F32) + b1[...])
    x1_ref[...] = x1
    _proj_out(x1, wkt, bk, wqt, bq, wvt, bv, wst, gb, k_ref, qv_ref, s_ref)


def _tc_mid_body(agg_ref, s_ref, xp_ref, wiht, biht, whht, bhht,
                 wkt, bk, wqt, bq, wvt, bv, wst, gb,
                 x2_ref, k_ref, qv_ref, s2_ref):
    xp = xp_ref[...]
    agg = agg_ref[0] + agg_ref[1]
    h = _elu(agg + s_ref[...])
    x2 = jnp.maximum(_gru_math(h, xp, wiht[...], biht[...], whht[...], bhht[...]), 0.0)
    x2_ref[...] = x2
    _proj_out(x2, wkt, bk, wqt, bq, wvt, bv, wst, gb, k_ref, qv_ref, s2_ref)


def _tc_post_body(agg_ref, s_ref, xp_ref, wiht, biht, whht, bhht, wgt, asrc,
                  x3_ref, xs_ref, as_ref):
    xp = xp_ref[...]
    agg = agg_ref[0] + agg_ref[1]
    h = _elu(agg + s_ref[...])
    x3 = jnp.maximum(_gru_math(h, xp, wiht[...], biht[...], whht[...], bhht[...]), 0.0)
    x3_ref[...] = x3
    xs = jnp.dot(x3, wgt[...], preferred_element_type=F32)
    xs_ref[...] = xs
    as_ref[...] = jnp.dot(xs, asrc[...], preferred_element_type=F32)


def _tc_gat_body(T, x3_ref, xs_ref, asrc_ref, bcol_ref, brow_ref,
                 wgt, adst, gbias, wiht, biht, whht, bhht, w2t, b2, y_ref):
    Bg = y_ref.shape[0]
    bcol = bcol_ref[...]
    brow = brow_ref[...]
    st_bool = bcol == lax.broadcasted_iota(jnp.int32, (1, Bg), 1)
    stf = st_bool.astype(F32)
    sf = (lax.broadcasted_iota(jnp.int32, (Bg, 1), 0) == brow).astype(F32)
    x3 = x3_ref[...]
    xs = xs_ref[...]
    a_src = asrc_ref[...]
    out = jnp.maximum(jnp.dot(sf, x3, preferred_element_type=F32), 0.0)
    for _ in range(T):
        od = jnp.dot(out, wgt[...], preferred_element_type=F32)
        a_dst = jnp.dot(od, adst[...], preferred_element_type=F32)
        alpha = _leaky(a_src + jnp.dot(stf, a_dst, preferred_element_type=F32))
        masked = jnp.where(st_bool, alpha, -3.0e38)
        amax = jnp.max(masked, axis=0, keepdims=True)
        ex = jnp.exp(alpha - jnp.dot(stf, amax.T, preferred_element_type=F32))
        den = jnp.dot(sf, ex, preferred_element_type=F32)
        att = ex / (jnp.dot(stf, den, preferred_element_type=F32) + 1e-16)
        m = jnp.dot(sf, att * xs, preferred_element_type=F32) + gbias[...]
        h = _elu(m)
        out = jnp.maximum(
            _gru_math(h, out, wiht[...], biht[...], whht[...], bhht[...]), 0.0)
    y_ref[...] = jnp.dot(out, w2t[...], preferred_element_type=F32) + b2[...]


def _row_spec(rb, d):
    return pl.BlockSpec((rb, d), lambda i: (i, 0))


def _full_spec(shape):
    nd = len(shape)
    return pl.BlockSpec(shape, lambda i: (0,) * nd)


def _make_edge_agg(N, H, E, C, CHW):
    NT = 16
    ZR = 40
    NCH = N // ZR
    NRR = (NCH + NT - 1) // NT
    mesh = plsc.VectorSubcoreMesh(core_axis_name="c", subcore_axis_name="s")

    assert E % C == 0

    @functools.partial(
        pl.kernel,
        out_type=jax.ShapeDtypeStruct((2 * N, H), F32),
        mesh=mesh,
        scratch_types=[
            pltpu.VMEM((C,), jnp.int32),
            pltpu.VMEM((C,), jnp.int32),
            pltpu.VMEM((C,), jnp.int32),
            pltpu.VMEM((C,), jnp.int32),
            pltpu.VMEM((C, H), F32),
            pltpu.VMEM((C, H), F32),
            pltpu.VMEM((C, 2 * H), F32),
            pltpu.VMEM((C, 2 * H), F32),
            pltpu.VMEM((C, H), F32),
            pltpu.VMEM((ZR, H), F32),
            pltpu.VMEM_SHARED((N, H), F32),
            pltpu.SemaphoreType.DMA,
            pltpu.SemaphoreType.DMA,
            pltpu.SemaphoreType.DMA,
            pltpu.SemaphoreType.DMA,
        ],
    )
    def edge_agg(k_hbm, qv_hbm, src_hbm, dst_hbm, out_hbm,
                 src_a, dst_a, src_b, dst_b, kr_a, kr_b, qv_a, qv_b,
                 msg, zbuf, acc, semk_a, semq_a, semk_b, semq_b):
        c = lax.axis_index("c")
        s = lax.axis_index("s")
        w = c * NT + s

        zero = jnp.zeros((16,), F32)

        def zrow(i, carry):
            for l in range(H // 16):
                zbuf[i, pl.ds(l * 16, 16)] = zero
            return carry

        lax.fori_loop(0, ZR, zrow, 0)
        for r in range(NRR):
            cid = s + NT * r

            @pl.when(cid < NCH)
            def _():
                pltpu.sync_copy(zbuf, acc.at[pl.ds(cid * ZR, ZR)])
        plsc.subcore_barrier()

        base = w * CHW * C
        nfull = jnp.clip(E - base, 0, CHW * C) // C

        bufs = ((src_a, dst_a, kr_a, qv_a, semk_a, semq_a),
                (src_b, dst_b, kr_b, qv_b, semk_b, semq_b))

        def fire(buf, j):
            src_c, dst_c, krows, qvrows, semk, semq = buf
            flat0 = base + j * C
            pltpu.sync_copy(src_hbm.at[pl.ds(flat0, C)], src_c)
            pltpu.sync_copy(dst_hbm.at[pl.ds(flat0, C)], dst_c)
            pltpu.async_copy(k_hbm.at[dst_c], krows, semk)
            pltpu.async_copy(qv_hbm.at[src_c], qvrows, semq)

        def process(cur, nxt, j):
            @pl.when(j + 1 < nfull)
            def _():
                fire(nxt, j + 1)
            src_c, dst_c, krows, qvrows, semk, semq = cur
            pltpu.make_async_copy(k_hbm.at[dst_c], krows, semk).wait()
            pltpu.make_async_copy(qv_hbm.at[src_c], qvrows, semq).wait()

            def edge(e0, ecarry):
                for u in range(4):
                    e = e0 * 4 + u
                    for l in range(H // 16):
                        kk = krows[e, pl.ds(l * 16, 16)]
                        qq = qvrows[e, pl.ds(l * 16, 16)]
                        vv = qvrows[e, pl.ds(H + l * 16, 16)]
                        g = 1.0 / (1.0 + jnp.exp(-(kk + qq)))
                        msg[e, pl.ds(l * 16, 16)] = g * vv
                return ecarry

            lax.fori_loop(0, C // 4, edge, 0)
            pltpu.sync_copy(msg, acc.at[dst_c], add=True)

        @pl.when(0 < nfull)
        def _():
            fire(bufs[0], 0)

        def chunk(j, carry):
            even = lax.rem(j, 2) == 0

            @pl.when(even)
            def _():
                process(bufs[0], bufs[1], j)

            @pl.when(jnp.logical_not(even))
            def _():
                process(bufs[1], bufs[0], j)

            return carry

        lax.fori_loop(0, nfull, chunk, 0)

        plsc.subcore_barrier()
        for r in range(NRR):
            cid = s + NT * r

            @pl.when(cid < NCH)
            def _():
                pltpu.sync_copy(acc.at[pl.ds(cid * ZR, ZR)], zbuf)
                pltpu.sync_copy(zbuf, out_hbm.at[pl.ds(c * N + cid * ZR, ZR)])

    return edge_agg


def kernel(x, edge_index, edge_attr, batch, params):
    N, IN = x.shape
    E = edge_index.shape[1]
    B = 256
    H = params['W1'].shape[0]
    T = 8
    RB = 2000
    NROWS = N // RB
    C = 32
    W = 32
    CHW = -(-E // (W * C))
    EPAD = W * CHW * C - E

    g1, g2 = params['g1'], params['g2']
    gat = params['gat']
    pad = jnp.zeros((EPAD,), jnp.int32)
    src2 = jnp.concatenate([edge_index[0], pad])
    dst2 = jnp.concatenate([edge_index[1], pad])
    bcol = batch.reshape(N, 1)
    brow = batch.reshape(1, N)

    rs = functools.partial(_row_spec, RB)
    wspecs = [_full_spec(s) for s in
              ((IN, H), (1, H), (H, H), (1, H), (H, H), (1, H), (H, H), (1, H),
               (H, H), (1, H))]
    proj_outspecs = [rs(H), rs(H), rs(2 * H), rs(H)]
    proj_outshapes = [jax.ShapeDtypeStruct((N, H), F32),
                      jax.ShapeDtypeStruct((N, H), F32),
                      jax.ShapeDtypeStruct((N, 2 * H), F32),
                      jax.ShapeDtypeStruct((N, H), F32)]
    x1, k1, qv1, s1 = pl.pallas_call(
        _tc_pre_body,
        grid=(NROWS,),
        in_specs=[rs(IN)] + wspecs,
        out_specs=proj_outspecs,
        out_shape=proj_outshapes,
    )(x, params['W1'].T, params['b1'].reshape(1, H),
      g1['Wk'].T, g1['bk'].reshape(1, H), g1['Wq'].T, g1['bq'].reshape(1, H),
      g1['Wv'].T, g1['bv'].reshape(1, H), g1['Wskip'].T, g1['bias'].reshape(1, H))

    edge_agg = _make_edge_agg(N, H, E, C, CHW)

    agg1 = edge_agg(k1, qv1, src2, dst2).reshape(2, N, H)

    gru0 = params['gru0']
    aggspec = pl.BlockSpec((2, RB, H), lambda i: (0, i, 0))
    gruspecs = [_full_spec(s) for s in
                ((H, 3 * H), (1, 3 * H), (H, 3 * H), (1, 3 * H))]
    x2, k2, qv2, s2 = pl.pallas_call(
        _tc_mid_body,
        grid=(NROWS,),
        in_specs=[aggspec, rs(H), rs(H)] + gruspecs + wspecs[2:],
        out_specs=proj_outspecs,
        out_shape=proj_outshapes,
    )(agg1, s1, x1,
      gru0['Wih'].T, gru0['bih'].reshape(1, 3 * H),
      gru0['Whh'].T, gru0['bhh'].reshape(1, 3 * H),
      g2['Wk'].T, g2['bk'].reshape(1, H), g2['Wq'].T, g2['bq'].reshape(1, H),
      g2['Wv'].T, g2['bv'].reshape(1, H), g2['Wskip'].T, g2['bias'].reshape(1, H))

    agg2 = edge_agg(k2, qv2, src2, dst2).reshape(2, N, H)

    gru1 = params['gru1']
    x3, xs, a_src = pl.pallas_call(
        _tc_post_body,
        grid=(NROWS,),
        in_specs=[aggspec, rs(H), rs(H)] + gruspecs +
                 [_full_spec((H, H)), _full_spec((H, 1))],
        out_specs=[rs(H), rs(H), rs(1)],
        out_shape=[jax.ShapeDtypeStruct((N, H), F32),
                   jax.ShapeDtypeStruct((N, H), F32),
                   jax.ShapeDtypeStruct((N, 1), F32)],
    )(agg2, s2, x2,
      gru1['Wih'].T, gru1['bih'].reshape(1, 3 * H),
      gru1['Whh'].T, gru1['bhh'].reshape(1, 3 * H),
      gat['Wg'].T, gat['att_src'].reshape(H, 1))

    mg = params['molgru']
    y = pl.pallas_call(
        functools.partial(_tc_gat_body, T),
        out_shape=jax.ShapeDtypeStruct((B, 1), F32),
    )(x3, xs, a_src, bcol, brow,
      gat['Wg'].T, gat['att_dst'].reshape(H, 1), gat['bias'].reshape(1, H),
      mg['Wih'].T, mg['bih'].reshape(1, 3 * H),
      mg['Whh'].T, mg['bhh'].reshape(1, 3 * H),
      params['W2'].T, params['b2'].reshape(1, 1))
    return y


# speedup vs baseline: 2.1176x; 1.0001x over previous
import functools

import jax
import jax.numpy as jnp
from jax import lax
from jax.experimental import pallas as pl
from jax.experimental.pallas import tpu as pltpu
from jax.experimental.pallas import tpu_sc as plsc

F32 = jnp.float32


def _sigm(x):
    return 1.0 / (1.0 + jnp.exp(-x))


def _leaky(x):
    return jnp.where(x > 0, x, 0.01 * x)


def _elu(x):
    return jnp.where(x > 0, x, jnp.exp(jnp.minimum(x, 0.0)) - 1.0)


def _gru_math(inp, hid, wiht, biht, whht, bhht):
    H = hid.shape[1]
    gi = jnp.dot(inp, wiht, preferred_element_type=F32) + biht
    gh = jnp.dot(hid, whht, preferred_element_type=F32) + bhht
    r = _sigm(gi[:, :H] + gh[:, :H])
    z = _sigm(gi[:, H:2 * H] + gh[:, H:2 * H])
    n = jnp.tanh(gi[:, 2 * H:] + r * gh[:, 2 * H:])
    return (1.0 - z) * n + z * hid


def _proj_out(x1, wkt, bk, wqt, bq, wvt, bv, wst, gb, k_ref, qv_ref, s_ref):
    H = k_ref.shape[1]
    k_ref[...] = jnp.dot(x1, wkt[...], preferred_element_type=F32) + bk[...]
    qv_ref[:, :H] = jnp.dot(x1, wqt[...], preferred_element_type=F32) + bq[...]
    qv_ref[:, H:] = jnp.dot(x1, wvt[...], preferred_element_type=F32) + bv[...]
    s_ref[...] = jnp.dot(x1, wst[...], preferred_element_type=F32) + gb[...]


def _tc_pre_body(x_ref, w1t, b1, wkt, bk, wqt, bq, wvt, bv, wst, gb,
                 x1_ref, k_ref, qv_ref, s_ref):
    x = x_ref[...]
    x1 = _leaky(jnp.dot(x, w1t[...], preferred_element_type=F32) + b1[...])
    x1_ref[...] = x1
    _proj_out(x1, wkt, bk, wqt, bq, wvt, bv, wst, gb, k_ref, qv_ref, s_ref)


def _tc_mid_body(agg_ref, s_ref, xp_ref, wiht, biht, whht, bhht,
                 wkt, bk, wqt, bq, wvt, bv, wst, gb,
                 x2_ref, k_ref, qv_ref, s2_ref):
    xp = xp_ref[...]
    agg = agg_ref[0] + agg_ref[1]
    h = _elu(agg + s_ref[...])
    x2 = jnp.maximum(_gru_math(h, xp, wiht[...], biht[...], whht[...], bhht[...]), 0.0)
    x2_ref[...] = x2
    _proj_out(x2, wkt, bk, wqt, bq, wvt, bv, wst, gb, k_ref, qv_ref, s2_ref)


def _tc_post_body(agg_ref, s_ref, xp_ref, wiht, biht, whht, bhht, wgt, asrc,
                  x3_ref, xs_ref, as_ref):
    xp = xp_ref[...]
    agg = agg_ref[0] + agg_ref[1]
    h = _elu(agg + s_ref[...])
    x3 = jnp.maximum(_gru_math(h, xp, wiht[...], biht[...], whht[...], bhht[...]), 0.0)
    x3_ref[...] = x3
    xs = jnp.dot(x3, wgt[...], preferred_element_type=F32)
    xs_ref[...] = xs
    as_ref[...] = jnp.dot(xs, asrc[...], preferred_element_type=F32)


def _tc_gat_body(T, x3_ref, xs_ref, asrc_ref, bcol_ref, brow_ref,
                 wgt, adst, gbias, wiht, biht, whht, bhht, w2t, b2, y_ref):
    Bg = y_ref.shape[0]
    bcol = bcol_ref[...]
    brow = brow_ref[...]
    st_bool = bcol == lax.broadcasted_iota(jnp.int32, (1, Bg), 1)
    stf = st_bool.astype(F32)
    sf = (lax.broadcasted_iota(jnp.int32, (Bg, 1), 0) == brow).astype(F32)
    x3 = x3_ref[...]
    xs = xs_ref[...]
    a_src = asrc_ref[...]
    out = jnp.maximum(jnp.dot(sf, x3, preferred_element_type=F32), 0.0)
    for _ in range(T):
        od = jnp.dot(out, wgt[...], preferred_element_type=F32)
        a_dst = jnp.dot(od, adst[...], preferred_element_type=F32)
        alpha = _leaky(a_src + jnp.dot(stf, a_dst, preferred_element_type=F32))
        masked = jnp.where(st_bool, alpha, -3.0e38)
        amax = jnp.max(masked, axis=0, keepdims=True)
        ex = jnp.exp(alpha - jnp.dot(stf, amax.T, preferred_element_type=F32))
        den = jnp.dot(sf, ex, preferred_element_type=F32)
        att = ex / (jnp.dot(stf, den, preferred_element_type=F32) + 1e-16)
        m = jnp.dot(sf, att * xs, preferred_element_type=F32) + gbias[...]
        h = _elu(m)
        out = jnp.maximum(
            _gru_math(h, out, wiht[...], biht[...], whht[...], bhht[...]), 0.0)
    y_ref[...] = jnp.dot(out, w2t[...], preferred_element_type=F32) + b2[...]


def _row_spec(rb, d):
    return pl.BlockSpec((rb, d), lambda i: (i, 0))


def _full_spec(shape):
    nd = len(shape)
    return pl.BlockSpec(shape, lambda i: (0,) * nd)


def _make_edge_agg(N, H, E, C, CHW):
    NT = 16
    ZR = 40
    NCH = N // ZR
    NRR = (NCH + NT - 1) // NT
    mesh = plsc.VectorSubcoreMesh(core_axis_name="c", subcore_axis_name="s")

    assert E % C == 0

    @functools.partial(
        pl.kernel,
        out_type=jax.ShapeDtypeStruct((2 * N, H), F32),
        mesh=mesh,
        scratch_types=[
            pltpu.VMEM((C,), jnp.int32),
            pltpu.VMEM((C,), jnp.int32),
            pltpu.VMEM((C,), jnp.int32),
            pltpu.VMEM((C,), jnp.int32),
            pltpu.VMEM((C, H), F32),
            pltpu.VMEM((C, H), F32),
            pltpu.VMEM((C, 2 * H), F32),
            pltpu.VMEM((C, 2 * H), F32),
            pltpu.VMEM((C, H), F32),
            pltpu.VMEM((ZR, H), F32),
            pltpu.VMEM_SHARED((N, H), F32),
            pltpu.SemaphoreType.DMA,
            pltpu.SemaphoreType.DMA,
            pltpu.SemaphoreType.DMA,
            pltpu.SemaphoreType.DMA,
        ],
    )
    def edge_agg(k_hbm, qv_hbm, src_hbm, dst_hbm, out_hbm,
                 src_a, dst_a, src_b, dst_b, kr_a, kr_b, qv_a, qv_b,
                 msg, zbuf, acc, semk_a, semq_a, semk_b, semq_b):
        c = lax.axis_index("c")
        s = lax.axis_index("s")
        w = c * NT + s

        zero = jnp.zeros((16,), F32)

        def zrow(i, carry):
            for l in range(H // 16):
                zbuf[i, pl.ds(l * 16, 16)] = zero
            return carry

        lax.fori_loop(0, ZR, zrow, 0)
        for r in range(NRR):
            cid = s + NT * r

            @pl.when(cid < NCH)
            def _():
                pltpu.sync_copy(zbuf, acc.at[pl.ds(cid * ZR, ZR)])
        plsc.subcore_barrier()

        base = w * CHW * C
        nfull = jnp.clip(E - base, 0, CHW * C) // C

        bufs = ((src_a, dst_a, kr_a, qv_a, semk_a, semq_a),
                (src_b, dst_b, kr_b, qv_b, semk_b, semq_b))

        def fire(buf, j):
            src_c, dst_c, krows, qvrows, semk, semq = buf
            flat0 = base + j * C
            pltpu.sync_copy(src_hbm.at[pl.ds(flat0, C)], src_c)
            pltpu.sync_copy(dst_hbm.at[pl.ds(flat0, C)], dst_c)
            pltpu.async_copy(k_hbm.at[dst_c], krows, semk)
            pltpu.async_copy(qv_hbm.at[src_c], qvrows, semq)

        def process(cur, nxt, j):
            @pl.when(j + 1 < nfull)
            def _():
                fire(nxt, j + 1)
            src_c, dst_c, krows, qvrows, semk, semq = cur
            pltpu.make_async_copy(k_hbm.at[dst_c], krows, semk).wait()
            pltpu.make_async_copy(qv_hbm.at[src_c], qvrows, semq).wait()

            def edge(e0, ecarry):
                for l in range(H // 16):
                    for u in range(4):
                        e = e0 * 4 + u
                        kk = krows[e, pl.ds(l * 16, 16)]
                        qq = qvrows[e, pl.ds(l * 16, 16)]
                        vv = qvrows[e, pl.ds(H + l * 16, 16)]
                        g = 1.0 / (1.0 + jnp.exp(-(kk + qq)))
                        msg[e, pl.ds(l * 16, 16)] = g * vv
                return ecarry

            lax.fori_loop(0, C // 4, edge, 0)
            pltpu.sync_copy(msg, acc.at[dst_c], add=True)

        @pl.when(0 < nfull)
        def _():
            fire(bufs[0], 0)

        def chunk(j, carry):
            even = lax.rem(j, 2) == 0

            @pl.when(even)
            def _():
                process(bufs[0], bufs[1], j)

            @pl.when(jnp.logical_not(even))
            def _():
                process(bufs[1], bufs[0], j)

            return carry

        lax.fori_loop(0, nfull, chunk, 0)

        plsc.subcore_barrier()
        for r in range(NRR):
            cid = s + NT * r

            @pl.when(cid < NCH)
            def _():
                pltpu.sync_copy(acc.at[pl.ds(cid * ZR, ZR)], zbuf)
                pltpu.sync_copy(zbuf, out_hbm.at[pl.ds(c * N + cid * ZR, ZR)])

    return edge_agg


def kernel(x, edge_index, edge_attr, batch, params):
    N, IN = x.shape
    E = edge_index.shape[1]
    B = 256
    H = params['W1'].shape[0]
    T = 8
    RB = 2000
    NROWS = N // RB
    C = 32
    W = 32
    CHW = -(-E // (W * C))
    EPAD = W * CHW * C - E

    g1, g2 = params['g1'], params['g2']
    gat = params['gat']
    pad = jnp.zeros((EPAD,), jnp.int32)
    src2 = jnp.concatenate([edge_index[0], pad])
    dst2 = jnp.concatenate([edge_index[1], pad])
    bcol = batch.reshape(N, 1)
    brow = batch.reshape(1, N)

    rs = functools.partial(_row_spec, RB)
    wspecs = [_full_spec(s) for s in
              ((IN, H), (1, H), (H, H), (1, H), (H, H), (1, H), (H, H), (1, H),
               (H, H), (1, H))]
    proj_outspecs = [rs(H), rs(H), rs(2 * H), rs(H)]
    proj_outshapes = [jax.ShapeDtypeStruct((N, H), F32),
                      jax.ShapeDtypeStruct((N, H), F32),
                      jax.ShapeDtypeStruct((N, 2 * H), F32),
                      jax.ShapeDtypeStruct((N, H), F32)]
    x1, k1, qv1, s1 = pl.pallas_call(
        _tc_pre_body,
        grid=(NROWS,),
        in_specs=[rs(IN)] + wspecs,
        out_specs=proj_outspecs,
        out_shape=proj_outshapes,
    )(x, params['W1'].T, params['b1'].reshape(1, H),
      g1['Wk'].T, g1['bk'].reshape(1, H), g1['Wq'].T, g1['bq'].reshape(1, H),
      g1['Wv'].T, g1['bv'].reshape(1, H), g1['Wskip'].T, g1['bias'].reshape(1, H))

    edge_agg = _make_edge_agg(N, H, E, C, CHW)

    agg1 = edge_agg(k1, qv1, src2, dst2).reshape(2, N, H)

    gru0 = params['gru0']
    aggspec = pl.BlockSpec((2, RB, H), lambda i: (0, i, 0))
    gruspecs = [_full_spec(s) for s in
                ((H, 3 * H), (1, 3 * H), (H, 3 * H), (1, 3 * H))]
    x2, k2, qv2, s2 = pl.pallas_call(
        _tc_mid_body,
        grid=(NROWS,),
        in_specs=[aggspec, rs(H), rs(H)] + gruspecs + wspecs[2:],
        out_specs=proj_outspecs,
        out_shape=proj_outshapes,
    )(agg1, s1, x1,
      gru0['Wih'].T, gru0['bih'].reshape(1, 3 * H),
      gru0['Whh'].T, gru0['bhh'].reshape(1, 3 * H),
      g2['Wk'].T, g2['bk'].reshape(1, H), g2['Wq'].T, g2['bq'].reshape(1, H),
      g2['Wv'].T, g2['bv'].reshape(1, H), g2['Wskip'].T, g2['bias'].reshape(1, H))

    agg2 = edge_agg(k2, qv2, src2, dst2).reshape(2, N, H)

    gru1 = params['gru1']
    x3, xs, a_src = pl.pallas_call(
        _tc_post_body,
        grid=(NROWS,),
        in_specs=[aggspec, rs(H), rs(H)] + gruspecs +
                 [_full_spec((H, H)), _full_spec((H, 1))],
        out_specs=[rs(H), rs(H), rs(1)],
        out_shape=[jax.ShapeDtypeStruct((N, H), F32),
                   jax.ShapeDtypeStruct((N, H), F32),
                   jax.ShapeDtypeStruct((N, 1), F32)],
    )(agg2, s2, x2,
      gru1['Wih'].T, gru1['bih'].reshape(1, 3 * H),
      gru1['Whh'].T, gru1['bhh'].reshape(1, 3 * H),
      gat['Wg'].T, gat['att_src'].reshape(H, 1))

    mg = params['molgru']
    y = pl.pallas_call(
        functools.partial(_tc_gat_body, T),
        out_shape=jax.ShapeDtypeStruct((B, 1), F32),
    )(x3, xs, a_src, bcol, brow,
      gat['Wg'].T, gat['att_dst'].reshape(H, 1), gat['bias'].reshape(1, H),
      mg['Wih'].T, mg['bih'].reshape(1, 3 * H),
      mg['Whh'].T, mg['bhh'].reshape(1, 3 * H),
      params['W2'].T, params['b2'].reshape(1, 1))
    return y
